# baseline (speedup 1.0000x reference)
import functools

import jax
import jax.numpy as jnp
from jax import lax
from jax.experimental import pallas as pl
from jax.experimental.pallas import tpu as pltpu
from jax.experimental.pallas import tpu_sc as plsc

N = 50000
K = 16
L = 16
NC, NS = 2, 16
NW = NC * NS
C = 160
CK = C * K
CPW = 1600
NP = NW * CPW
NG = C // L
NCHUNK = CPW // C
GSUB = 128
NSUB = CK // GSUB

_MAGIC = 0x5F3759DF
_SWEEPS = 5
_GAMMA = 5.828427124746189
_CH8 = 0.9238795325112867
_SH8 = 0.3826834323650898


def _rsqrt(x, iters=3):
    i = plsc.bitcast(x, jnp.int32)
    y = plsc.bitcast(_MAGIC - lax.shift_right_logical(i, 1), jnp.float32)
    for _ in range(iters):
        y = y * (1.5 - 0.5 * x * y * y)
    return y


def _sqrt(x, iters=3):
    return x * _rsqrt(x, iters)


def _recip(x, iters=3):
    y = _rsqrt(x * x, iters)
    return x * y * y


def _cross(a, b):
    return [a[1] * b[2] - a[2] * b[1],
            a[2] * b[0] - a[0] * b[2],
            a[0] * b[1] - a[1] * b[0]]


def _body(xyz12, nbr, wm, nn, area, alcw, out,
          idx_v, rows_v, wt_v, nn_v, area_v, own_v, alcw_v, out_v, shared,
          semL, semG, semG2, semO):
    ii = lax.iota(jnp.int32, L)
    zi = jnp.zeros((L,), jnp.int32)
    sid = lax.axis_index("s")
    wid = sid * NC + lax.axis_index("c")
    base = wid * CPW

    SROWS = NP // NS
    pltpu.sync_copy(xyz12.at[pl.ds(sid * SROWS, SROWS)],
                    shared.at[pl.ds(sid * SROWS, SROWS)])
    plsc.subcore_barrier()

    pltpu.sync_copy(alcw, alcw_v)
    al = alcw_v[pl.ds(0, L)]
    cw = alcw_v[pl.ds(L, L)]
    one_m_al = 1.0 - al

    ccol = [jnp.full((L,), c, jnp.int32) for c in range(K)]

    def issue_linear(g, b):
        vb = base + g * C
        pltpu.async_copy(nbr.at[pl.ds(vb * K, CK)], idx_v.at[b], semL)
        pltpu.async_copy(wm.at[pl.ds(vb, C)], wt_v.at[b], semL)
        pltpu.async_copy(nn.at[pl.ds(vb, C)], nn_v.at[b], semL)
        pltpu.async_copy(area.at[pl.ds(vb, C)], area_v.at[b], semL)
        pltpu.async_copy(xyz12.at[pl.ds(vb, C)], own_v.at[b], semL)

    def wait_linear(b):
        pltpu.make_async_copy(nbr.at[pl.ds(0, CK)], idx_v.at[b], semL).wait()
        pltpu.make_async_copy(wm.at[pl.ds(0, C)], wt_v.at[b], semL).wait()
        pltpu.make_async_copy(nn.at[pl.ds(0, C)], nn_v.at[b], semL).wait()
        pltpu.make_async_copy(area.at[pl.ds(0, C)], area_v.at[b], semL).wait()
        pltpu.make_async_copy(xyz12.at[pl.ds(0, C)], own_v.at[b], semL).wait()

    def issue_gather(b):
        for j in range(NSUB):
            pltpu.async_copy(
                shared.at[idx_v.at[b, pl.ds(j * GSUB, GSUB)]],
                rows_v.at[b, pl.ds(j * GSUB, GSUB)], semG)

    def wait_gather(b):
        pltpu.make_async_copy(xyz12.at[pl.ds(0, CK)], rows_v.at[b], semG).wait()

    issue_linear(0, 0)
    wait_linear(0)
    issue_gather(0)
    if NCHUNK > 1:
        issue_linear(1, 1)

    def chunk(g, carry):
        vb = base + g * C
        b = lax.rem(g, 2)
        bn = lax.rem(g + 1, 2)

        @pl.when(g + 1 < NCHUNK)
        def _prefetch():
            wait_linear(bn)
            issue_gather(bn)

        wait_gather(b)
        b_vec = zi + b

        @pl.when(g >= 2)
        def _drain_out():
            pltpu.make_async_copy(out_v.at[b], out.at[pl.ds(0, C)], semO).wait()

        def group(gi, gcarry):
            vrow = gi * L + ii
            row0 = gi * (L * K) + ii * K
            nnv = plsc.load_gather(nn_v, [b_vec, vrow])
            areav = plsc.load_gather(area_v, [b_vec, vrow])
            o = [plsc.load_gather(own_v, [b_vec, vrow, ccol[c]]) for c in range(6)]

            z = jnp.zeros((L,), jnp.float32)
            S = [[z, z, z], [z, z, z], [z, z, z]]
            P = [[z, z, z], [z, z, z], [z, z, z]]
            q = [z, z, z]
            for k in range(K - 1):
                rk = row0 + k
                wk = plsc.load_gather(wt_v, [b_vec, vrow, ccol[k]])
                wk = jnp.where(nnv > k, wk, 0.0)
                nb = [plsc.load_gather(rows_v, [b_vec, rk, ccol[c]])
                      for c in range(6)]
                d1 = [o[c] - nb[c] for c in range(3)]
                d2 = [o[3 + c] - nb[3 + c] for c in range(3)]
                wd1 = [wk * d1[i] for i in range(3)]
                for i in range(3):
                    for j in range(3):
                        S[i][j] = S[i][j] + wd1[i] * d2[j]
                for i in range(3):
                    for j in range(i, 3):
                        P[i][j] = P[i][j] + wd1[i] * d1[j]
                for i in range(3):
                    q[i] = q[i] + (wk * d2[i]) * d2[i]
            for i in range(3):
                for j in range(i):
                    P[i][j] = P[j][i]

            A = [[z, z, z], [z, z, z], [z, z, z]]
            for i in range(3):
                for j in range(i, 3):
                    acc = S[0][i] * S[0][j] + S[1][i] * S[1][j] + S[2][i] * S[2][j]
                    A[i][j] = acc
                    A[j][i] = acc
            inv_tr = _recip(A[0][0] + A[1][1] + A[2][2] + 1e-30)
            for i in range(3):
                for j in range(3):
                    A[i][j] = A[i][j] * inv_tr
            one = jnp.full((L,), 1.0, jnp.float32)
            V = [[one, z, z], [z, one, z], [z, z, one]]
            for _sweep in range(_SWEEPS):
                for (p, qq) in ((0, 1), (0, 2), (1, 2)):
                    app = A[p][p]
                    aqq = A[qq][qq]
                    apq = A[p][qq]
                    ch = 2.0 * (aqq - app)
                    sh = apq
                    use = _GAMMA * sh * sh < ch * ch
                    om = _rsqrt(ch * ch + sh * sh + 1e-38, 2)
                    chn = jnp.where(use, om * ch, _CH8)
                    shn = jnp.where(use, om * sh, _SH8)
                    cj = chn * chn - shn * shn
                    sj = 2.0 * shn * chn
                    r = 3 - p - qq
                    arp = A[r][p]
                    arq = A[r][qq]
                    nrp = cj * arp - sj * arq
                    nrq = sj * arp + cj * arq
                    A[r][p] = nrp
                    A[p][r] = nrp
                    A[r][qq] = nrq
                    A[qq][r] = nrq
                    sc_ = sj * cj
                    cc_ = cj * cj
                    ss_ = sj * sj
                    app_n = cc_ * app - 2.0 * sc_ * apq + ss_ * aqq
                    aqq_n = ss_ * app + 2.0 * sc_ * apq + cc_ * aqq
                    apq_n = sc_ * (app - aqq) + (cc_ - ss_) * apq
                    A[p][p] = app_n
                    A[qq][qq] = aqq_n
                    A[p][qq] = apq_n
                    A[qq][p] = apq_n
                    for i in range(3):
                        vp = V[i][p]
                        vq = V[i][qq]
                        V[i][p] = cj * vp - sj * vq
                        V[i][qq] = sj * vp + cj * vq

            lam = [A[0][0], A[1][1], A[2][2]]
            cols = [[V[0][j], V[1][j], V[2][j]] for j in range(3)]
            for (i, j) in ((0, 1), (0, 2), (1, 2)):
                sw = lam[i] < lam[j]
                li = lam[i]
                lam[i] = jnp.where(sw, lam[j], li)
                lam[j] = jnp.where(sw, li, lam[j])
                for c in range(3):
                    ci = cols[i][c]
                    cols[i][c] = jnp.where(sw, cols[j][c], ci)
                    cols[j][c] = jnp.where(sw, ci, cols[j][c])
            v1, v2 = cols[0], cols[1]
            v3 = _cross(v1, v2)

            b1 = [S[i][0] * v1[0] + S[i][1] * v1[1] + S[i][2] * v1[2]
                  for i in range(3)]
            b2 = [S[i][0] * v2[0] + S[i][1] * v2[1] + S[i][2] * v2[2]
                  for i in range(3)]
            u1s = _rsqrt(b1[0] * b1[0] + b1[1] * b1[1] + b1[2] * b1[2] + 1e-30)
            u1 = [b1[i] * u1s for i in range(3)]
            dot = u1[0] * b2[0] + u1[1] * b2[1] + u1[2] * b2[2]
            b2p = [b2[i] - dot * u1[i] for i in range(3)]
            u2s = _rsqrt(b2p[0] * b2p[0] + b2p[1] * b2p[1] + b2p[2] * b2p[2]
                         + 1e-30)
            u2 = [b2p[i] * u2s for i in range(3)]
            u3 = _cross(u1, u2)

            Rb = [[al * (v1[i] * u1[j] + v2[i] * u2[j] + v3[i] * u3[j])
                   for j in range(3)] for i in range(3)]
            for i in range(3):
                Rb[i][i] = Rb[i][i] + one_m_al

            scale = areav * cw
            for i in range(3):
                t2 = Rb[i][0] * S[0][i] + Rb[i][1] * S[1][i] + Rb[i][2] * S[2][i]
                rp = [Rb[i][0] * P[0][j] + Rb[i][1] * P[1][j] + Rb[i][2] * P[2][j]
                      for j in range(3)]
                t3 = rp[0] * Rb[i][0] + rp[1] * Rb[i][1] + rp[2] * Rb[i][2]
                e = (q[i] - 2.0 * t2 + t3) * scale
                plsc.store_scatter(out_v, [b_vec, vrow, ccol[i]], e)
            return gcarry

        lax.fori_loop(0, NG, group, 0, unroll=2)
        pltpu.async_copy(out_v.at[b], out.at[pl.ds(vb, C)], semO)

        @pl.when(g + 2 < NCHUNK)
        def _stage_ahead():
            issue_linear(g + 2, b)
        return carry

    lax.fori_loop(0, NCHUNK, chunk, 0)
    pltpu.make_async_copy(out_v.at[0], out.at[pl.ds(0, C)], semO).wait()
    pltpu.make_async_copy(out_v.at[1], out.at[pl.ds(0, C)], semO).wait()


_mesh = plsc.VectorSubcoreMesh(core_axis_name="c", subcore_axis_name="s")

_carap_sc = functools.partial(
    pl.kernel,
    mesh=_mesh,
    compiler_params=pltpu.CompilerParams(
        needs_layout_passes=False, use_tc_tiling_on_sc=False),
    out_type=jax.ShapeDtypeStruct((NP, 3), jnp.float32),
    scratch_types=[
        pltpu.VMEM((2, CK), jnp.int32),
        pltpu.VMEM((2, CK, 8), jnp.float32),
        pltpu.VMEM((2, C, K), jnp.float32),
        pltpu.VMEM((2, C), jnp.int32),
        pltpu.VMEM((2, C), jnp.float32),
        pltpu.VMEM((2, C, 8), jnp.float32),
        pltpu.VMEM((2 * L,), jnp.float32),
        pltpu.VMEM((2, C, 3), jnp.float32),
        pltpu.VMEM_SHARED((NP, 8), jnp.float32),
        pltpu.SemaphoreType.DMA,
        pltpu.SemaphoreType.DMA,
        pltpu.SemaphoreType.DMA,
        pltpu.SemaphoreType.DMA,
    ],
)(_body)


def kernel(xyz1, xyz2, neighborList, numNeighbors, weightMatrix, alpha, area,
           carapWeight):
    x1 = xyz1.reshape(N, 3).astype(jnp.float32)
    x2 = xyz2.reshape(N, 3).astype(jnp.float32)
    xyz12 = jnp.concatenate([x1, x2, jnp.zeros((N, 2), jnp.float32)], axis=1)
    xyz12 = jnp.pad(xyz12, ((0, NP - N), (0, 0)))
    nbr = jnp.pad(neighborList.reshape(N, K), ((0, NP - N), (0, 0)))
    wm = jnp.pad(weightMatrix.reshape(N, K).astype(jnp.float32),
                 ((0, NP - N), (0, 0)))
    nnum = jnp.pad(numNeighbors.reshape(N), (0, NP - N))
    ar = jnp.pad(area.reshape(N).astype(jnp.float32), (0, NP - N))
    alcw = jnp.concatenate([
        jnp.broadcast_to(jnp.asarray(alpha, jnp.float32), (L,)),
        jnp.broadcast_to(jnp.asarray(carapWeight, jnp.float32), (L,)),
    ])
    outp = _carap_sc(xyz12, nbr.reshape(NP * K), wm, nnum, ar, alcw)
    return outp[:N].reshape(1, N, 3)


# speedup vs baseline: 1.2708x; 1.2708x over previous
import functools

import jax
import jax.numpy as jnp
from jax import lax
from jax.experimental import pallas as pl
from jax.experimental.pallas import tpu as pltpu
from jax.experimental.pallas import tpu_sc as plsc

N = 50000
K = 16
L = 16
NC, NS = 2, 16
NW = NC * NS
C = 160
CK = C * K
CPW = 1600
NP = NW * CPW
NG = C // L
NCHUNK = CPW // C
GSUB = 128
NSUB = CK // GSUB

_MAGIC = 0x5F3759DF
_SWEEPS = 4
_GAMMA = 5.828427124746189
_CH8 = 0.9238795325112867
_SH8 = 0.3826834323650898


def _rsqrt(x, iters=3):
    i = plsc.bitcast(x, jnp.int32)
    y = plsc.bitcast(_MAGIC - lax.shift_right_logical(i, 1), jnp.float32)
    for _ in range(iters):
        y = y * (1.5 - 0.5 * x * y * y)
    return y


def _sqrt(x, iters=3):
    return x * _rsqrt(x, iters)


def _recip(x, iters=3):
    y = _rsqrt(x * x, iters)
    return x * y * y


def _cross(a, b):
    return [a[1] * b[2] - a[2] * b[1],
            a[2] * b[0] - a[0] * b[2],
            a[0] * b[1] - a[1] * b[0]]


def _body(xyz12, nbr, wm, nn, area, alcw, out,
          idx_v, rows_v, wt_v, nn_v, area_v, own_v, alcw_v, out_v, shared,
          semL, semG, semG2, semO):
    ii = lax.iota(jnp.int32, L)
    zi = jnp.zeros((L,), jnp.int32)
    sid = lax.axis_index("s")
    wid = sid * NC + lax.axis_index("c")
    base = wid * CPW

    SROWS = NP // NS
    pltpu.sync_copy(xyz12.at[pl.ds(sid * SROWS, SROWS)],
                    shared.at[pl.ds(sid * SROWS, SROWS)])
    plsc.subcore_barrier()

    pltpu.sync_copy(alcw, alcw_v)
    al = alcw_v[pl.ds(0, L)]
    cw = alcw_v[pl.ds(L, L)]
    one_m_al = 1.0 - al

    ccol = [jnp.full((L,), c, jnp.int32) for c in range(K)]

    def issue_linear(g, b):
        vb = base + g * C
        pltpu.async_copy(nbr.at[pl.ds(vb * K, CK)], idx_v.at[b], semL)
        pltpu.async_copy(wm.at[pl.ds(vb, C)], wt_v.at[b], semL)
        pltpu.async_copy(nn.at[pl.ds(vb, C)], nn_v.at[b], semL)
        pltpu.async_copy(area.at[pl.ds(vb, C)], area_v.at[b], semL)
        pltpu.async_copy(xyz12.at[pl.ds(vb, C)], own_v.at[b], semL)

    def wait_linear(b):
        pltpu.make_async_copy(nbr.at[pl.ds(0, CK)], idx_v.at[b], semL).wait()
        pltpu.make_async_copy(wm.at[pl.ds(0, C)], wt_v.at[b], semL).wait()
        pltpu.make_async_copy(nn.at[pl.ds(0, C)], nn_v.at[b], semL).wait()
        pltpu.make_async_copy(area.at[pl.ds(0, C)], area_v.at[b], semL).wait()
        pltpu.make_async_copy(xyz12.at[pl.ds(0, C)], own_v.at[b], semL).wait()

    def issue_gather(b):
        for j in range(NSUB):
            pltpu.async_copy(
                shared.at[idx_v.at[b, pl.ds(j * GSUB, GSUB)]],
                rows_v.at[b, pl.ds(j * GSUB, GSUB)], semG)

    def wait_gather(b):
        pltpu.make_async_copy(xyz12.at[pl.ds(0, CK)], rows_v.at[b], semG).wait()

    issue_linear(0, 0)
    wait_linear(0)
    issue_gather(0)
    if NCHUNK > 1:
        issue_linear(1, 1)

    def chunk(g, carry):
        vb = base + g * C
        b = lax.rem(g, 2)
        bn = lax.rem(g + 1, 2)

        @pl.when(g + 1 < NCHUNK)
        def _prefetch():
            wait_linear(bn)
            issue_gather(bn)

        wait_gather(b)
        b_vec = zi + b

        @pl.when(g >= 2)
        def _drain_out():
            pltpu.make_async_copy(out_v.at[b], out.at[pl.ds(0, C)], semO).wait()

        def group(gi, gcarry):
            vrow = gi * L + ii
            row0 = gi * (L * K) + ii * K
            nnv = plsc.load_gather(nn_v, [b_vec, vrow])
            areav = plsc.load_gather(area_v, [b_vec, vrow])
            o = [plsc.load_gather(own_v, [b_vec, vrow, ccol[c]]) for c in range(6)]

            z = jnp.zeros((L,), jnp.float32)
            S = [[z, z, z], [z, z, z], [z, z, z]]
            P = [[z, z, z], [z, z, z], [z, z, z]]
            q = [z, z, z]
            for k in range(K - 1):
                rk = row0 + k
                wk = plsc.load_gather(wt_v, [b_vec, vrow, ccol[k]])
                wk = jnp.where(nnv > k, wk, 0.0)
                nb = [plsc.load_gather(rows_v, [b_vec, rk, ccol[c]])
                      for c in range(6)]
                d1 = [o[c] - nb[c] for c in range(3)]
                d2 = [o[3 + c] - nb[3 + c] for c in range(3)]
                wd1 = [wk * d1[i] for i in range(3)]
                for i in range(3):
                    for j in range(3):
                        S[i][j] = S[i][j] + wd1[i] * d2[j]
                for i in range(3):
                    for j in range(i, 3):
                        P[i][j] = P[i][j] + wd1[i] * d1[j]
                for i in range(3):
                    q[i] = q[i] + (wk * d2[i]) * d2[i]
            for i in range(3):
                for j in range(i):
                    P[i][j] = P[j][i]

            A = [[z, z, z], [z, z, z], [z, z, z]]
            for i in range(3):
                for j in range(i, 3):
                    acc = S[0][i] * S[0][j] + S[1][i] * S[1][j] + S[2][i] * S[2][j]
                    A[i][j] = acc
                    A[j][i] = acc
            inv_tr = _recip(A[0][0] + A[1][1] + A[2][2] + 1e-30)
            for i in range(3):
                for j in range(3):
                    A[i][j] = A[i][j] * inv_tr
            one = jnp.full((L,), 1.0, jnp.float32)
            V = [[one, z, z], [z, one, z], [z, z, one]]
            for _sweep in range(_SWEEPS):
                for (p, qq) in ((0, 1), (0, 2), (1, 2)):
                    app = A[p][p]
                    aqq = A[qq][qq]
                    apq = A[p][qq]
                    ch = 2.0 * (aqq - app)
                    sh = apq
                    use = _GAMMA * sh * sh < ch * ch
                    om = _rsqrt(ch * ch + sh * sh + 1e-38, 2)
                    chn = jnp.where(use, om * ch, _CH8)
                    shn = jnp.where(use, om * sh, _SH8)
                    cj = chn * chn - shn * shn
                    sj = 2.0 * shn * chn
                    r = 3 - p - qq
                    arp = A[r][p]
                    arq = A[r][qq]
                    nrp = cj * arp - sj * arq
                    nrq = sj * arp + cj * arq
                    A[r][p] = nrp
                    A[p][r] = nrp
                    A[r][qq] = nrq
                    A[qq][r] = nrq
                    sc_ = sj * cj
                    cc_ = cj * cj
                    ss_ = sj * sj
                    app_n = cc_ * app - 2.0 * sc_ * apq + ss_ * aqq
                    aqq_n = ss_ * app + 2.0 * sc_ * apq + cc_ * aqq
                    apq_n = sc_ * (app - aqq) + (cc_ - ss_) * apq
                    A[p][p] = app_n
                    A[qq][qq] = aqq_n
                    A[p][qq] = apq_n
                    A[qq][p] = apq_n
                    for i in range(3):
                        vp = V[i][p]
                        vq = V[i][qq]
                        V[i][p] = cj * vp - sj * vq
                        V[i][qq] = sj * vp + cj * vq

            lam = [A[0][0], A[1][1], A[2][2]]
            cols = [[V[0][j], V[1][j], V[2][j]] for j in range(3)]
            for (i, j) in ((0, 1), (0, 2), (1, 2)):
                sw = lam[i] < lam[j]
                li = lam[i]
                lam[i] = jnp.where(sw, lam[j], li)
                lam[j] = jnp.where(sw, li, lam[j])
                for c in range(3):
                    ci = cols[i][c]
                    cols[i][c] = jnp.where(sw, cols[j][c], ci)
                    cols[j][c] = jnp.where(sw, ci, cols[j][c])
            v1, v2 = cols[0], cols[1]
            v3 = _cross(v1, v2)

            b1 = [S[i][0] * v1[0] + S[i][1] * v1[1] + S[i][2] * v1[2]
                  for i in range(3)]
            b2 = [S[i][0] * v2[0] + S[i][1] * v2[1] + S[i][2] * v2[2]
                  for i in range(3)]
            u1s = _rsqrt(b1[0] * b1[0] + b1[1] * b1[1] + b1[2] * b1[2] + 1e-30)
            u1 = [b1[i] * u1s for i in range(3)]
            dot = u1[0] * b2[0] + u1[1] * b2[1] + u1[2] * b2[2]
            b2p = [b2[i] - dot * u1[i] for i in range(3)]
            u2s = _rsqrt(b2p[0] * b2p[0] + b2p[1] * b2p[1] + b2p[2] * b2p[2]
                         + 1e-30)
            u2 = [b2p[i] * u2s for i in range(3)]
            u3 = _cross(u1, u2)

            Rb = [[al * (v1[i] * u1[j] + v2[i] * u2[j] + v3[i] * u3[j])
                   for j in range(3)] for i in range(3)]
            for i in range(3):
                Rb[i][i] = Rb[i][i] + one_m_al

            scale = areav * cw
            for i in range(3):
                t2 = Rb[i][0] * S[0][i] + Rb[i][1] * S[1][i] + Rb[i][2] * S[2][i]
                rp = [Rb[i][0] * P[0][j] + Rb[i][1] * P[1][j] + Rb[i][2] * P[2][j]
                      for j in range(3)]
                t3 = rp[0] * Rb[i][0] + rp[1] * Rb[i][1] + rp[2] * Rb[i][2]
                e = (q[i] - 2.0 * t2 + t3) * scale
                plsc.store_scatter(out_v, [b_vec, vrow, ccol[i]], e)
            return gcarry

        lax.fori_loop(0, NG, group, 0)
        pltpu.async_copy(out_v.at[b], out.at[pl.ds(vb, C)], semO)

        @pl.when(g + 2 < NCHUNK)
        def _stage_ahead():
            issue_linear(g + 2, b)
        return carry

    lax.fori_loop(0, NCHUNK, chunk, 0)
    pltpu.make_async_copy(out_v.at[0], out.at[pl.ds(0, C)], semO).wait()
    pltpu.make_async_copy(out_v.at[1], out.at[pl.ds(0, C)], semO).wait()


_mesh = plsc.VectorSubcoreMesh(core_axis_name="c", subcore_axis_name="s")

_carap_sc = functools.partial(
    pl.kernel,
    mesh=_mesh,
    compiler_params=pltpu.CompilerParams(
        needs_layout_passes=False, use_tc_tiling_on_sc=False),
    out_type=jax.ShapeDtypeStruct((NP, 3), jnp.float32),
    scratch_types=[
        pltpu.VMEM((2, CK), jnp.int32),
        pltpu.VMEM((2, CK, 8), jnp.float32),
        pltpu.VMEM((2, C, K), jnp.float32),
        pltpu.VMEM((2, C), jnp.int32),
        pltpu.VMEM((2, C), jnp.float32),
        pltpu.VMEM((2, C, 8), jnp.float32),
        pltpu.VMEM((2 * L,), jnp.float32),
        pltpu.VMEM((2, C, 3), jnp.float32),
        pltpu.VMEM_SHARED((NP, 8), jnp.float32),
        pltpu.SemaphoreType.DMA,
        pltpu.SemaphoreType.DMA,
        pltpu.SemaphoreType.DMA,
        pltpu.SemaphoreType.DMA,
    ],
)(_body)


def kernel(xyz1, xyz2, neighborList, numNeighbors, weightMatrix, alpha, area,
           carapWeight):
    x1 = xyz1.reshape(N, 3).astype(jnp.float32)
    x2 = xyz2.reshape(N, 3).astype(jnp.float32)
    xyz12 = jnp.concatenate([x1, x2, jnp.zeros((N, 2), jnp.float32)], axis=1)
    xyz12 = jnp.pad(xyz12, ((0, NP - N), (0, 0)))
    nbr = jnp.pad(neighborList.reshape(N, K), ((0, NP - N), (0, 0)))
    wm = jnp.pad(weightMatrix.reshape(N, K).astype(jnp.float32),
                 ((0, NP - N), (0, 0)))
    nnum = jnp.pad(numNeighbors.reshape(N), (0, NP - N))
    ar = jnp.pad(area.reshape(N).astype(jnp.float32), (0, NP - N))
    alcw = jnp.concatenate([
        jnp.broadcast_to(jnp.asarray(alpha, jnp.float32), (L,)),
        jnp.broadcast_to(jnp.asarray(carapWeight, jnp.float32), (L,)),
    ])
    outp = _carap_sc(xyz12, nbr.reshape(NP * K), wm, nnum, ar, alcw)
    return outp[:N].reshape(1, N, 3)
